# Initial kernel scaffold; baseline (speedup 1.0000x reference)
import jax
import jax.numpy as jnp
from jax.experimental import pallas as pl


def kernel(x, edge_index, W1, b1, W2, b2, W3, b3, W4, b4, g1, be1, g2, be2):
    raise NotImplementedError("write your pallas kernel here")


# speedup vs baseline: 5.7765x; 5.7765x over previous
import functools

import jax
import jax.numpy as jnp
import numpy as np
from jax import lax
from jax.experimental import pallas as pl
from jax.experimental.pallas import tpu as pltpu
from jax.experimental.pallas import tpu_sc as plsc

N = 10000
E = 320000
D = 128

NC = 2
NS = 16
NW = NC * NS

CHUNK = 128
NCHUNK = E // CHUNK
BASE_CHUNKS = NCHUNK // NW
EXTRA = NCHUNK - BASE_CHUNKS * NW
ROWS_PER_SUB = 624
TAIL_ROWS = N - NS * ROWS_PER_SUB


def _seg_sum_body(src_hbm, dst_hbm, x_hbm, zeros_hbm, out_hbm,
                  idx_s, idx_d, rows, acc, sem):
    c = lax.axis_index("c")
    s = lax.axis_index("s")
    w = s * NC + c

    @pl.when(c == 0)
    def _():
        pltpu.sync_copy(x_hbm.at[pl.ds(s * ROWS_PER_SUB, ROWS_PER_SUB)],
                        acc.at[pl.ds(s * ROWS_PER_SUB, ROWS_PER_SUB)])

    @pl.when(c == 1)
    def _():
        pltpu.sync_copy(zeros_hbm,
                        acc.at[pl.ds(s * ROWS_PER_SUB, ROWS_PER_SUB)])

    @pl.when((s == 0) & (c == 0))
    def _():
        pltpu.sync_copy(x_hbm.at[pl.ds(NS * ROWS_PER_SUB, TAIL_ROWS)],
                        acc.at[pl.ds(NS * ROWS_PER_SUB, TAIL_ROWS)])

    @pl.when((s == 0) & (c == 1))
    def _():
        pltpu.sync_copy(zeros_hbm.at[pl.ds(0, TAIL_ROWS)],
                        acc.at[pl.ds(NS * ROWS_PER_SUB, TAIL_ROWS)])

    plsc.subcore_barrier()

    def do_chunk(ci):
        pltpu.sync_copy(src_hbm.at[ci], idx_s)
        pltpu.async_copy(x_hbm.at[idx_s], rows, sem).wait()
        pltpu.sync_copy(dst_hbm.at[ci], idx_d)
        pltpu.sync_copy(rows, acc.at[idx_d], add=True)

    def loop_body(i, carry):
        do_chunk(w * BASE_CHUNKS + i)
        return carry

    lax.fori_loop(0, BASE_CHUNKS, loop_body, 0)

    @pl.when(w < EXTRA)
    def _():
        do_chunk(NW * BASE_CHUNKS + w)

    plsc.subcore_barrier()
    pltpu.sync_copy(acc.at[pl.ds(s * ROWS_PER_SUB, ROWS_PER_SUB)],
                    out_hbm.at[c, pl.ds(s * ROWS_PER_SUB, ROWS_PER_SUB)])

    @pl.when(s == 0)
    def _():
        pltpu.sync_copy(acc.at[pl.ds(NS * ROWS_PER_SUB, TAIL_ROWS)],
                        out_hbm.at[c, pl.ds(NS * ROWS_PER_SUB, TAIL_ROWS)])


def _seg_sum(src2d, dst2d, x, zeros):
    mesh = plsc.VectorSubcoreMesh(core_axis_name="c", subcore_axis_name="s",
                                  num_cores=NC, num_subcores=NS)
    f = pl.kernel(
        _seg_sum_body,
        out_type=jax.ShapeDtypeStruct((NC, N, D), jnp.float32),
        mesh=mesh,
        scratch_types=[
            pltpu.VMEM((CHUNK,), jnp.int32),
            pltpu.VMEM((CHUNK,), jnp.int32),
            pltpu.VMEM((CHUNK, D), jnp.float32),
            pltpu.VMEM_SHARED((N, D), jnp.float32),
            pltpu.SemaphoreType.DMA,
        ],
    )
    return f(src2d, dst2d, x, zeros)


def _mlp_bn_body(p_ref, Wa_ref, ba_ref, Wb_ref, bb_ref, g_ref, be_ref,
                 out_ref):
    h = p_ref[0] + p_ref[1]
    h = jnp.maximum(
        jnp.dot(h, Wa_ref[...], preferred_element_type=jnp.float32) + ba_ref[...], 0.0)
    h = jnp.dot(h, Wb_ref[...], preferred_element_type=jnp.float32) + bb_ref[...]
    mu = jnp.mean(h, axis=0, keepdims=True)
    var = jnp.mean((h - mu) * (h - mu), axis=0, keepdims=True)
    h = (h - mu) * lax.rsqrt(var + 1e-5) * g_ref[...] + be_ref[...]
    out_ref[...] = jnp.maximum(h, 0.0)


def _mlp_bn_res_body(p_ref, Wa_ref, ba_ref, Wb_ref, bb_ref, g_ref, be_ref,
                     x0_ref, out_ref):
    h = p_ref[0] + p_ref[1]
    h = jnp.maximum(
        jnp.dot(h, Wa_ref[...], preferred_element_type=jnp.float32) + ba_ref[...], 0.0)
    h = jnp.dot(h, Wb_ref[...], preferred_element_type=jnp.float32) + bb_ref[...]
    mu = jnp.mean(h, axis=0, keepdims=True)
    var = jnp.mean((h - mu) * (h - mu), axis=0, keepdims=True)
    h = (h - mu) * lax.rsqrt(var + 1e-5) * g_ref[...] + be_ref[...]
    out_ref[...] = (jnp.maximum(h, 0.0) + x0_ref[...]) * np.float32(1.0 / np.sqrt(2.0))


def _mlp_bn(p, Wa, ba, Wb, bb, g, be):
    return pl.pallas_call(
        _mlp_bn_body,
        out_shape=jax.ShapeDtypeStruct((N, D), jnp.float32),
    )(p, Wa, ba.reshape(1, D), Wb, bb.reshape(1, D), g.reshape(1, D),
      be.reshape(1, D))


def _mlp_bn_res(p, Wa, ba, Wb, bb, g, be, x0):
    return pl.pallas_call(
        _mlp_bn_res_body,
        out_shape=jax.ShapeDtypeStruct((N, D), jnp.float32),
    )(p, Wa, ba.reshape(1, D), Wb, bb.reshape(1, D), g.reshape(1, D),
      be.reshape(1, D), x0)


def kernel(x, edge_index, W1, b1, W2, b2, W3, b3, W4, b4, g1, be1, g2, be2):
    src2d = edge_index[0].astype(jnp.int32).reshape(NCHUNK, CHUNK)
    dst2d = edge_index[1].astype(jnp.int32).reshape(NCHUNK, CHUNK)
    zeros = jnp.zeros((ROWS_PER_SUB, D), jnp.float32)

    p1 = _seg_sum(src2d, dst2d, x, zeros)
    h1 = _mlp_bn(p1, W1, b1, W2, b2, g1, be1)
    p2 = _seg_sum(src2d, dst2d, h1, zeros)
    return _mlp_bn_res(p2, W3, b3, W4, b4, g2, be2, x)
